# baseline (speedup 1.0000x reference)
import functools

import jax
import jax.numpy as jnp
from jax import lax
from jax.experimental import pallas as pl
from jax.experimental.pallas import tpu as pltpu
from jax.experimental.pallas import tpu_sc as plsc

VOCAB = 457
EMB = 64
GATHER_UNIT = 128


@functools.cache
def _build(batch: int, seq: int):
    info = plsc.get_sparse_core_info()
    nw = info.num_cores * info.num_subcores
    rows_per_w = batch // nw
    assert rows_per_w * nw == batch
    splits = []
    off = 0
    while off < seq:
        n = min(GATHER_UNIT, seq - off)
        splits.append((off, n))
        off += n

    mesh = plsc.VectorSubcoreMesh(core_axis_name="c", subcore_axis_name="s")
    out_t = jax.ShapeDtypeStruct((batch, seq, EMB), jnp.float32)

    @functools.partial(
        pl.kernel,
        mesh=mesh,
        out_type=out_t,
        scratch_types=[
            pltpu.VMEM((2, seq), jnp.int32),
            pltpu.VMEM((2, seq, EMB), jnp.float32),
            pltpu.VMEM_SHARED((VOCAB, EMB), jnp.float32),
            pltpu.SemaphoreType.DMA,
            pltpu.SemaphoreType.DMA,
            pltpu.SemaphoreType.DMA,
            pltpu.SemaphoreType.DMA,
            pltpu.SemaphoreType.DMA,
            pltpu.SemaphoreType.DMA,
        ],
        compiler_params=pltpu.CompilerParams(use_tc_tiling_on_sc=False),
    )
    def k(w_tbl, idx_hbm, out_hbm, idx_v, rows_v, w_s,
          si0, si1, sg0, sg1, so0, so1):
        wid = lax.axis_index("s") * info.num_cores + lax.axis_index("c")
        sem_i, sem_g, sem_o = [si0, si1], [sg0, sg1], [so0, so1]

        @pl.when(lax.axis_index("s") == 0)
        def _():
            pltpu.sync_copy(w_tbl, w_s)
        plsc.subcore_barrier()

        def start_idx(b, row):
            pltpu.async_copy(idx_hbm.at[row], idx_v.at[b], sem_i[b])

        def gather_cps(b):
            return [
                pltpu.make_async_copy(
                    w_s.at[idx_v.at[b].at[pl.ds(off, n)]],
                    rows_v.at[b].at[pl.ds(off, n)],
                    sem_g[b])
                for off, n in splits
            ]

        def wait_store(b):
            pltpu.make_async_copy(rows_v.at[b], out_hbm.at[0], sem_o[b]).wait()

        base = wid * rows_per_w

        for b in range(2):
            start_idx(b, base + b)

        def pair_body(p, carry):
            for b in range(2):
                j = 2 * p + b
                pltpu.make_async_copy(
                    idx_hbm.at[0], idx_v.at[b], sem_i[b]).wait()

                @pl.when(j >= 2)
                def _():
                    wait_store(b)
                for cp in gather_cps(b):
                    cp.start()

            for b in range(2):
                j = 2 * p + b
                for cp in gather_cps(b):
                    cp.wait()
                pltpu.async_copy(rows_v.at[b], out_hbm.at[base + j], sem_o[b])

                @pl.when(j + 2 < rows_per_w)
                def _():
                    start_idx(b, base + j + 2)
            return carry

        lax.fori_loop(0, rows_per_w // 2, pair_body, 0)

        for b in range(2):
            wait_store(b)

    return k


def kernel(seqs, exps, W_seq, W_exp):
    b, s = seqs.shape
    gather = _build(b, s)
    o_seq = gather(W_seq, seqs.astype(jnp.int32))
    o_exp = gather(W_exp, exps.astype(jnp.int32))
    one = 1.0 + 0.0 * W_seq[0, 0]
    return (o_seq * one, o_exp * one)


# speedup vs baseline: 1.0640x; 1.0640x over previous
import functools

import jax
import jax.numpy as jnp
from jax import lax
from jax.experimental import pallas as pl
from jax.experimental.pallas import tpu as pltpu
from jax.experimental.pallas import tpu_sc as plsc

VOCAB = 457
EMB = 64
VOCAB_P = 464
EMB_P = 128
SEQ_P = 512
GU = 128
G_PER_CHUNK = 2
CHUNK = GU * G_PER_CHUNK
BB = 16


@functools.cache
def _build_sc(batch: int, seq: int):
    info = plsc.get_sparse_core_info()
    nw = info.num_cores * info.num_subcores
    rows_per_w = batch // nw
    assert rows_per_w * nw == batch
    chunks_per_row = SEQ_P // CHUNK

    mesh = plsc.VectorSubcoreMesh(core_axis_name="c", subcore_axis_name="s")
    out_t = jax.ShapeDtypeStruct((batch, SEQ_P, EMB_P), jnp.float32)

    @functools.partial(
        pl.kernel,
        mesh=mesh,
        out_type=[out_t, out_t],
        scratch_types=[
            pltpu.VMEM((GU,), jnp.int32),
            pltpu.VMEM((GU,), jnp.int32),
            pltpu.VMEM((GU,), jnp.int32),
            pltpu.VMEM((GU,), jnp.int32),
            pltpu.VMEM((CHUNK, EMB_P), jnp.float32),
            pltpu.VMEM((CHUNK, EMB_P), jnp.float32),
            pltpu.VMEM_SHARED((VOCAB_P, EMB_P), jnp.float32),
            pltpu.VMEM_SHARED((VOCAB_P, EMB_P), jnp.float32),
            pltpu.SemaphoreType.DMA,
            pltpu.SemaphoreType.DMA,
            pltpu.SemaphoreType.DMA,
            pltpu.SemaphoreType.DMA,
            pltpu.SemaphoreType.DMA,
            pltpu.SemaphoreType.DMA,
        ],
        compiler_params=pltpu.CompilerParams(use_tc_tiling_on_sc=True),
    )
    def k(w_seq, w_exp, seq_idx, exp_idx, o_seq, o_exp,
          i00, i01, i10, i11, rows0, rows1,
          w_seq_s, w_exp_s, si0, si1, sg0, sg1, so0, so1):
        wid = lax.axis_index("s") * info.num_cores + lax.axis_index("c")
        idx_v = [[i00, i01], [i10, i11]]
        rows_v = [rows0, rows1]
        sem_i, sem_g, sem_o = [si0, si1], [sg0, sg1], [so0, so1]

        @pl.when(lax.axis_index("s") == 0)
        def _():
            pltpu.sync_copy(w_seq, w_seq_s)
            pltpu.sync_copy(w_exp, w_exp_s)
        plsc.subcore_barrier()

        def start_idx(idx_hbm, b, c):
            irow = c * G_PER_CHUNK
            for i in range(G_PER_CHUNK):
                pltpu.async_copy(idx_hbm.at[irow + i], idx_v[b][i], sem_i[b])

        def wait_idx(idx_hbm, b):
            for i in range(G_PER_CHUNK):
                pltpu.make_async_copy(
                    idx_hbm.at[0], idx_v[b][i], sem_i[b]).wait()

        def gather_cps(w_s, b):
            return [
                pltpu.make_async_copy(
                    w_s.at[idx_v[b][i]],
                    rows_v[b].at[pl.ds(i * GU, GU)],
                    sem_g[b])
                for i in range(G_PER_CHUNK)
            ]

        def start_store(out_hbm, b, c):
            row = c // chunks_per_row
            half = c % chunks_per_row
            pltpu.async_copy(
                rows_v[b],
                out_hbm.at[row].at[pl.ds(half * CHUNK, CHUNK)],
                sem_o[b])

        def wait_store(out_hbm, b):
            pltpu.make_async_copy(
                rows_v[b], out_hbm.at[0].at[pl.ds(0, CHUNK)], sem_o[b]).wait()

        n_chunks = rows_per_w * chunks_per_row

        def do_table(idx_hbm, w_s, out_hbm):
            cbase = wid * n_chunks

            for b in range(2):
                start_idx(idx_hbm, b, cbase + b)

            def pair_body(p, carry):
                for b in range(2):
                    j = 2 * p + b
                    wait_idx(idx_hbm, b)

                    @pl.when(j >= 2)
                    def _():
                        wait_store(out_hbm, b)
                    for cp in gather_cps(w_s, b):
                        cp.start()

                for b in range(2):
                    j = 2 * p + b
                    for cp in gather_cps(w_s, b):
                        cp.wait()
                    start_store(out_hbm, b, cbase + j)

                    @pl.when(j + 2 < n_chunks)
                    def _():
                        start_idx(idx_hbm, b, cbase + j + 2)
                return carry

            lax.fori_loop(0, n_chunks // 2, pair_body, 0)

            for b in range(2):
                wait_store(out_hbm, b)

        do_table(seq_idx, w_seq_s, o_seq)
        do_table(exp_idx, w_exp_s, o_exp)

    return k


def _depad_body(a_ref, b_ref, oa_ref, ob_ref):
    _, s, e = oa_ref.shape
    oa_ref[...] = a_ref[:, :s, :e]
    ob_ref[...] = b_ref[:, :s, :e]


@functools.cache
def _build_depad(batch: int, seq: int):
    in_spec = pl.BlockSpec((BB, SEQ_P, EMB_P), lambda i: (i, 0, 0))
    out_spec = pl.BlockSpec((BB, seq, EMB), lambda i: (i, 0, 0))
    out_t = jax.ShapeDtypeStruct((batch, seq, EMB), jnp.float32)
    return pl.pallas_call(
        _depad_body,
        grid=(batch // BB,),
        in_specs=[in_spec, in_spec],
        out_specs=[out_spec, out_spec],
        out_shape=[out_t, out_t],
    )


def kernel(seqs, exps, W_seq, W_exp):
    b, s = seqs.shape
    w_seq_p = jnp.pad(W_seq, ((0, VOCAB_P - VOCAB), (0, EMB_P - EMB)))
    w_exp_p = jnp.pad(W_exp, ((0, VOCAB_P - VOCAB), (0, EMB_P - EMB)))
    seq_i = jnp.pad(seqs.astype(jnp.int32), ((0, 0), (0, SEQ_P - s)))
    exp_i = jnp.pad(exps.astype(jnp.int32), ((0, 0), (0, SEQ_P - s)))
    seq_i = seq_i.reshape(b * SEQ_P // GU, GU)
    exp_i = exp_i.reshape(b * SEQ_P // GU, GU)
    p_seq, p_exp = _build_sc(b, s)(w_seq_p, w_exp_p, seq_i, exp_i)
    return tuple(_build_depad(b, s)(p_seq, p_exp))


# speedup vs baseline: 1.4281x; 1.3422x over previous
import functools

import jax
import jax.numpy as jnp
from jax import lax
from jax.experimental import pallas as pl
from jax.experimental.pallas import tpu as pltpu
from jax.experimental.pallas import tpu_sc as plsc

VOCAB = 457
EMB = 64
GATHER_UNIT = 128


@functools.cache
def _build(batch: int, seq: int):
    info = plsc.get_sparse_core_info()
    nw = info.num_cores * info.num_subcores
    rows_per_w = batch // nw
    assert rows_per_w * nw == batch
    splits = []
    off = 0
    while off < seq:
        n = min(GATHER_UNIT, seq - off)
        splits.append((off, n))
        off += n

    mesh = plsc.VectorSubcoreMesh(core_axis_name="c", subcore_axis_name="s")
    out_t = jax.ShapeDtypeStruct((batch, seq, EMB), jnp.float32)

    @functools.partial(
        pl.kernel,
        mesh=mesh,
        out_type=out_t,
        scratch_types=[
            pltpu.VMEM((2, seq), jnp.int32),
            pltpu.VMEM((2, seq, EMB), jnp.float32),
            pltpu.VMEM_SHARED((VOCAB, EMB), jnp.float32),
            pltpu.SemaphoreType.DMA,
            pltpu.SemaphoreType.DMA,
            pltpu.SemaphoreType.DMA,
            pltpu.SemaphoreType.DMA,
            pltpu.SemaphoreType.DMA,
            pltpu.SemaphoreType.DMA,
        ],
        compiler_params=pltpu.CompilerParams(use_tc_tiling_on_sc=False),
    )
    def k(w_tbl, idx_hbm, out_hbm, idx_v, rows_v, w_s,
          si0, si1, sg0, sg1, so0, so1):
        wid = lax.axis_index("s") * info.num_cores + lax.axis_index("c")
        sem_i, sem_g, sem_o = [si0, si1], [sg0, sg1], [so0, so1]

        @pl.when(lax.axis_index("s") == 0)
        def _():
            pltpu.sync_copy(w_tbl, w_s)
        plsc.subcore_barrier()

        def start_idx(b, row):
            pltpu.async_copy(idx_hbm.at[row], idx_v.at[b], sem_i[b])

        def gather_cps(b):
            return [
                pltpu.make_async_copy(
                    w_s.at[idx_v.at[b].at[pl.ds(off, n)]],
                    rows_v.at[b].at[pl.ds(off, n)],
                    sem_g[b])
                for off, n in splits
            ]

        def wait_store(b):
            pltpu.make_async_copy(rows_v.at[b], out_hbm.at[0], sem_o[b]).wait()

        base = wid * rows_per_w

        for b in range(2):
            start_idx(b, base + b)

        def pair_body(p, carry):
            for b in range(2):
                j = 2 * p + b
                pltpu.make_async_copy(
                    idx_hbm.at[0], idx_v.at[b], sem_i[b]).wait()

                @pl.when(j >= 2)
                def _():
                    wait_store(b)
                for cp in gather_cps(b):
                    cp.start()

            for b in range(2):
                j = 2 * p + b
                for cp in gather_cps(b):
                    cp.wait()
                pltpu.async_copy(rows_v.at[b], out_hbm.at[base + j], sem_o[b])

                @pl.when(j + 2 < rows_per_w)
                def _():
                    start_idx(b, base + j + 2)
            return carry

        lax.fori_loop(0, rows_per_w // 2, pair_body, 0)

        for b in range(2):
            wait_store(b)

    return k


def kernel(seqs, exps, W_seq, W_exp):
    b, s = seqs.shape
    gather = _build(b, s)
    o_seq = gather(W_seq, seqs.astype(jnp.int32))
    o_exp = gather(W_exp, exps.astype(jnp.int32))
    one = 1.0 + 0.0 * W_seq[0, 0]
    return (o_seq * one, o_exp)


# speedup vs baseline: 1.6794x; 1.1760x over previous
import functools

import jax
import jax.numpy as jnp
from jax import lax
from jax.experimental import pallas as pl
from jax.experimental.pallas import tpu as pltpu
from jax.experimental.pallas import tpu_sc as plsc

VOCAB = 457
EMB = 64
GATHER_UNIT = 128


@functools.cache
def _build(batch: int, seq: int):
    info = plsc.get_sparse_core_info()
    nw = info.num_cores * info.num_subcores
    rows_per_w = batch // nw
    assert rows_per_w * nw == batch
    splits = []
    off = 0
    while off < seq:
        n = min(GATHER_UNIT, seq - off)
        splits.append((off, n))
        off += n

    mesh = plsc.VectorSubcoreMesh(core_axis_name="c", subcore_axis_name="s")
    out_t = jax.ShapeDtypeStruct((batch, seq, EMB), jnp.float32)

    @functools.partial(
        pl.kernel,
        mesh=mesh,
        out_type=[out_t, out_t],
        scratch_types=[
            pltpu.VMEM((2, seq), jnp.int32),
            pltpu.VMEM((2, seq, EMB), jnp.float32),
            pltpu.VMEM_SHARED((VOCAB, EMB), jnp.float32),
            pltpu.VMEM_SHARED((VOCAB, EMB), jnp.float32),
            pltpu.SemaphoreType.DMA,
            pltpu.SemaphoreType.DMA,
            pltpu.SemaphoreType.DMA,
            pltpu.SemaphoreType.DMA,
            pltpu.SemaphoreType.DMA,
            pltpu.SemaphoreType.DMA,
        ],
        compiler_params=pltpu.CompilerParams(use_tc_tiling_on_sc=False),
    )
    def k(w_seq, w_exp, seq_idx, exp_idx, o_seq, o_exp, idx_v, rows_v,
          w_seq_s, w_exp_s, si0, si1, sg0, sg1, so0, so1):
        wid = lax.axis_index("s") * info.num_cores + lax.axis_index("c")
        sem_i, sem_g, sem_o = [si0, si1], [sg0, sg1], [so0, so1]

        @pl.when(lax.axis_index("s") == 0)
        def _():
            pltpu.sync_copy(w_seq, w_seq_s)
            pltpu.sync_copy(w_exp, w_exp_s)
        plsc.subcore_barrier()

        def start_idx(idx_hbm, b, row):
            pltpu.async_copy(idx_hbm.at[row], idx_v.at[b], sem_i[b])

        def gather_cps(w_s, b):
            return [
                pltpu.make_async_copy(
                    w_s.at[idx_v.at[b].at[pl.ds(off, n)]],
                    rows_v.at[b].at[pl.ds(off, n)],
                    sem_g[b])
                for off, n in splits
            ]

        def wait_store(out_hbm, b):
            pltpu.make_async_copy(rows_v.at[b], out_hbm.at[0], sem_o[b]).wait()

        def do_table(idx_hbm, w_s, out_hbm):
            base = wid * rows_per_w

            for b in range(2):
                start_idx(idx_hbm, b, base + b)

            def pair_body(p, carry):
                for b in range(2):
                    j = 2 * p + b
                    pltpu.make_async_copy(
                        idx_hbm.at[0], idx_v.at[b], sem_i[b]).wait()

                    @pl.when(j >= 2)
                    def _():
                        wait_store(out_hbm, b)
                    for cp in gather_cps(w_s, b):
                        cp.start()

                for b in range(2):
                    j = 2 * p + b
                    for cp in gather_cps(w_s, b):
                        cp.wait()
                    pltpu.async_copy(rows_v.at[b], out_hbm.at[base + j], sem_o[b])

                    @pl.when(j + 2 < rows_per_w)
                    def _():
                        start_idx(idx_hbm, b, base + j + 2)
                return carry

            lax.fori_loop(0, rows_per_w // 2, pair_body, 0)

            for b in range(2):
                wait_store(out_hbm, b)

        do_table(seq_idx, w_seq_s, o_seq)
        do_table(exp_idx, w_exp_s, o_exp)

    return k


def kernel(seqs, exps, W_seq, W_exp):
    b, s = seqs.shape
    o_seq, o_exp = _build(b, s)(
        W_seq, W_exp, seqs.astype(jnp.int32), exps.astype(jnp.int32))
    return (o_seq, o_exp)
